# baseline (speedup 1.0000x reference)
import functools

import jax
import jax.numpy as jnp
from jax import lax
from jax.experimental import pallas as pl
from jax.experimental.pallas import tpu as pltpu
from jax.experimental.pallas import tpu_sc as plsc

K = 2
SCALE = 2.5
B, H, E, I, SI = 32, 1024, 64, 512, 2048


def _routing_body(x_ref, rw_ref, tki_ref, tkw_ref):
    x = x_ref[...]
    logits = jnp.dot(x, rw_ref[...], preferred_element_type=jnp.float32)
    m = jnp.max(logits, axis=1, keepdims=True)
    p = jnp.exp(logits - m)
    scores = p / jnp.sum(p, axis=1, keepdims=True)

    e_iota = jax.lax.broadcasted_iota(jnp.int32, (B, E), 1)
    m1 = jnp.max(scores, axis=1, keepdims=True)
    i1 = jnp.min(jnp.where(scores == m1, e_iota, E), axis=1, keepdims=True)
    sc2 = jnp.where(e_iota == i1, -jnp.inf, scores)
    m2 = jnp.max(sc2, axis=1, keepdims=True)
    i2 = jnp.min(jnp.where(sc2 == m2, e_iota, E), axis=1, keepdims=True)

    denom = m1 + m2 + 1e-20
    w1 = m1 / denom * SCALE
    w2 = m2 / denom * SCALE
    tki_ref[...] = jnp.concatenate([i1, i2], axis=1)
    tkw_ref[...] = jnp.concatenate([w1, w2], axis=1)


def _sc_compact_body(tki_hbm, ids_hbm, nact_hbm, tki_v, ids_v, nact_v):
    c = lax.axis_index("c")
    s = lax.axis_index("s")

    @pl.when((c == 0) & (s == 0))
    def _():
        pltpu.sync_copy(tki_hbm, tki_v)
        iota = lax.broadcasted_iota(jnp.int32, (16,), 0)
        zero = jnp.zeros((16,), jnp.int32)

        act = [zero, zero, zero, zero]
        for cs in range(B * K // 16):
            sv = tki_v[pl.ds(16 * cs, 16)]
            for j in range(16):
                s_j = jnp.sum(jnp.where(iota == j, sv, 0))
                for ce in range(E // 16):
                    hit = (iota + 16 * ce) == s_j
                    act[ce] = jnp.where(hit, 1, act[ce])

        carry = jnp.int32(0)
        last = jnp.int32(-1)
        cmp = [zero, zero, zero, zero]
        for e in range(E):
            a_e = jnp.sum(jnp.where(iota == (e % 16), act[e // 16], 0))
            sel = a_e > 0
            for cj in range(E // 16):
                put = ((iota + 16 * cj) == carry) & sel
                cmp[cj] = jnp.where(put, e, cmp[cj])
            carry = carry + a_e
            last = jnp.where(sel, e, last)

        for cj in range(E // 16):
            jv = iota + 16 * cj
            ids_v[pl.ds(16 * cj, 16)] = jnp.where(jv < carry, cmp[cj], last)
        nact_v[...] = zero + carry
        pltpu.sync_copy(ids_v, ids_hbm)
        pltpu.sync_copy(nact_v, nact_hbm)


def _expert_body(ids_ref, nact_ref, x_ref, gw_ref, uw_ref, dw_ref,
                 tki_ref, tkw_ref, sgw_ref, suw_ref, sdw_ref, out_ref):
    i = pl.program_id(0)

    @pl.when(i == 0)
    def _shared():
        x = x_ref[...].astype(jnp.bfloat16)
        sg = jnp.dot(x, sgw_ref[...].astype(jnp.bfloat16),
                     preferred_element_type=jnp.float32)
        su = jnp.dot(x, suw_ref[...].astype(jnp.bfloat16),
                     preferred_element_type=jnp.float32)
        act = (sg * jax.lax.logistic(sg) * su).astype(jnp.bfloat16)
        out_ref[...] = jnp.dot(act, sdw_ref[...].astype(jnp.bfloat16),
                               preferred_element_type=jnp.float32)

    @pl.when(i < nact_ref[0])
    def _expert():
        e = ids_ref[i]
        x = x_ref[...].astype(jnp.bfloat16)
        g = jnp.dot(x, gw_ref[0].astype(jnp.bfloat16),
                    preferred_element_type=jnp.float32)
        u = jnp.dot(x, uw_ref[0].astype(jnp.bfloat16),
                    preferred_element_type=jnp.float32)
        a = (g * jax.lax.logistic(g) * u).astype(jnp.bfloat16)
        y = jnp.dot(a, dw_ref[0].astype(jnp.bfloat16),
                    preferred_element_type=jnp.float32)
        coef = jnp.sum(jnp.where(tki_ref[...] == e, tkw_ref[...], 0.0),
                       axis=1, keepdims=True)
        out_ref[...] += y * coef


@jax.jit
def kernel(hidden_states, router_w, e_score_correction_bias, gate_w, up_w,
           down_w, shared_gate_w, shared_up_w, shared_down_w):
    del e_score_correction_bias
    x = hidden_states.reshape(B, H)

    tki, tkw = pl.pallas_call(
        _routing_body,
        out_shape=(
            jax.ShapeDtypeStruct((B, K), jnp.int32),
            jax.ShapeDtypeStruct((B, K), jnp.float32),
        ),
    )(x, router_w)

    sc_compact = functools.partial(
        pl.kernel,
        out_type=(
            jax.ShapeDtypeStruct((E,), jnp.int32),
            jax.ShapeDtypeStruct((16,), jnp.int32),
        ),
        mesh=plsc.VectorSubcoreMesh(core_axis_name="c", subcore_axis_name="s"),
        compiler_params=pltpu.CompilerParams(needs_layout_passes=False),
        scratch_types=[
            pltpu.VMEM((B * K,), jnp.int32),
            pltpu.VMEM((E,), jnp.int32),
            pltpu.VMEM((16,), jnp.int32),
        ],
    )(_sc_compact_body)
    ids, nact16 = sc_compact(tki.reshape(B * K))
    nact = nact16[:1]

    grid_spec = pltpu.PrefetchScalarGridSpec(
        num_scalar_prefetch=2,
        grid=(E,),
        in_specs=[
            pl.BlockSpec((B, H), lambda i, ids, nact: (0, 0)),
            pl.BlockSpec((1, H, I), lambda i, ids, nact: (ids[i], 0, 0)),
            pl.BlockSpec((1, H, I), lambda i, ids, nact: (ids[i], 0, 0)),
            pl.BlockSpec((1, I, H), lambda i, ids, nact: (ids[i], 0, 0)),
            pl.BlockSpec((B, K), lambda i, ids, nact: (0, 0)),
            pl.BlockSpec((B, K), lambda i, ids, nact: (0, 0)),
            pl.BlockSpec((H, SI), lambda i, ids, nact: (0, 0)),
            pl.BlockSpec((H, SI), lambda i, ids, nact: (0, 0)),
            pl.BlockSpec((SI, H), lambda i, ids, nact: (0, 0)),
        ],
        out_specs=pl.BlockSpec((B, H), lambda i, ids, nact: (0, 0)),
    )
    out = pl.pallas_call(
        _expert_body,
        grid_spec=grid_spec,
        out_shape=jax.ShapeDtypeStruct((B, H), jnp.float32),
        compiler_params=pltpu.CompilerParams(
            dimension_semantics=("arbitrary",),
        ),
    )(ids, nact, x, gate_w, up_w, down_w,
      tki, tkw, shared_gate_w, shared_up_w, shared_down_w)

    return out.reshape(B, 1, H)


# speedup vs baseline: 1.0136x; 1.0136x over previous
import functools

import jax
import jax.numpy as jnp
from jax import lax
from jax.experimental import pallas as pl
from jax.experimental.pallas import tpu as pltpu
from jax.experimental.pallas import tpu_sc as plsc

K = 2
SCALE = 2.5
B, H, E, I, SI = 32, 1024, 64, 512, 2048


def _routing_body(x_ref, rw_ref, tki_ref, tkw_ref):
    x = x_ref[...]
    logits = jnp.dot(x, rw_ref[...], preferred_element_type=jnp.float32)
    m = jnp.max(logits, axis=1, keepdims=True)
    p = jnp.exp(logits - m)
    scores = p / jnp.sum(p, axis=1, keepdims=True)

    e_iota = jax.lax.broadcasted_iota(jnp.int32, (B, E), 1)
    m1 = jnp.max(scores, axis=1, keepdims=True)
    i1 = jnp.min(jnp.where(scores == m1, e_iota, E), axis=1, keepdims=True)
    sc2 = jnp.where(e_iota == i1, -jnp.inf, scores)
    m2 = jnp.max(sc2, axis=1, keepdims=True)
    i2 = jnp.min(jnp.where(sc2 == m2, e_iota, E), axis=1, keepdims=True)

    denom = m1 + m2 + 1e-20
    w1 = m1 / denom * SCALE
    w2 = m2 / denom * SCALE
    tki_ref[...] = jnp.concatenate([i1, i2], axis=1)
    tkw_ref[...] = jnp.concatenate([w1, w2], axis=1)


def _sc_compact_body(tki_hbm, ids_hbm, nact_hbm, tki_v, ids_v, nact_v):
    c = lax.axis_index("c")
    s = lax.axis_index("s")

    @pl.when((c == 0) & (s == 0))
    def _():
        pltpu.sync_copy(tki_hbm, tki_v)
        iota = lax.broadcasted_iota(jnp.int32, (16,), 0)
        zero = jnp.zeros((16,), jnp.int32)

        act = [zero, zero, zero, zero]
        for cs in range(B * K // 16):
            sv = tki_v[pl.ds(16 * cs, 16)]
            for j in range(16):
                s_j = jnp.sum(jnp.where(iota == j, sv, 0))
                for ce in range(E // 16):
                    hit = (iota + 16 * ce) == s_j
                    act[ce] = jnp.where(hit, 1, act[ce])

        carry = jnp.int32(0)
        last = jnp.int32(-1)
        cmp = [zero, zero, zero, zero]
        for e in range(E):
            a_e = jnp.sum(jnp.where(iota == (e % 16), act[e // 16], 0))
            sel = a_e > 0
            for cj in range(E // 16):
                put = ((iota + 16 * cj) == carry) & sel
                cmp[cj] = jnp.where(put, e, cmp[cj])
            carry = carry + a_e
            last = jnp.where(sel, e, last)

        for cj in range(E // 16):
            jv = iota + 16 * cj
            ids_v[pl.ds(16 * cj, 16)] = jnp.where(jv < carry, cmp[cj], last)
        nact_v[...] = zero + carry
        pltpu.sync_copy(ids_v, ids_hbm)
        pltpu.sync_copy(nact_v, nact_hbm)


def _shared_body(x_ref, sgw_ref, suw_ref, sdw_ref, out_ref):
    x = x_ref[...].astype(jnp.bfloat16)
    sg = jnp.dot(x, sgw_ref[...].astype(jnp.bfloat16),
                 preferred_element_type=jnp.float32)
    su = jnp.dot(x, suw_ref[...].astype(jnp.bfloat16),
                 preferred_element_type=jnp.float32)
    act = (sg * jax.lax.logistic(sg) * su).astype(jnp.bfloat16)
    out_ref[...] = jnp.dot(act, sdw_ref[...].astype(jnp.bfloat16),
                           preferred_element_type=jnp.float32)


def _expert_body(ids_ref, nact_ref, x_ref, gw_ref, uw_ref, dw_ref,
                 tki_ref, tkw_ref, shared_ref, out_ref):
    i = pl.program_id(0)

    @pl.when(i == 0)
    def _init():
        out_ref[...] = shared_ref[...]

    @pl.when(i < nact_ref[0])
    def _expert():
        e = ids_ref[i]
        x = x_ref[...].astype(jnp.bfloat16)
        g = jnp.dot(x, gw_ref[0].astype(jnp.bfloat16),
                    preferred_element_type=jnp.float32)
        u = jnp.dot(x, uw_ref[0].astype(jnp.bfloat16),
                    preferred_element_type=jnp.float32)
        a = (g * jax.lax.logistic(g) * u).astype(jnp.bfloat16)
        y = jnp.dot(a, dw_ref[0].astype(jnp.bfloat16),
                    preferred_element_type=jnp.float32)
        coef = jnp.sum(jnp.where(tki_ref[...] == e, tkw_ref[...], 0.0),
                       axis=1, keepdims=True)
        out_ref[...] += y * coef


@jax.jit
def kernel(hidden_states, router_w, e_score_correction_bias, gate_w, up_w,
           down_w, shared_gate_w, shared_up_w, shared_down_w):
    del e_score_correction_bias
    x = hidden_states.reshape(B, H)

    tki, tkw = pl.pallas_call(
        _routing_body,
        out_shape=(
            jax.ShapeDtypeStruct((B, K), jnp.int32),
            jax.ShapeDtypeStruct((B, K), jnp.float32),
        ),
    )(x, router_w)

    sc_compact = functools.partial(
        pl.kernel,
        out_type=(
            jax.ShapeDtypeStruct((E,), jnp.int32),
            jax.ShapeDtypeStruct((16,), jnp.int32),
        ),
        mesh=plsc.VectorSubcoreMesh(core_axis_name="c", subcore_axis_name="s"),
        compiler_params=pltpu.CompilerParams(needs_layout_passes=False),
        scratch_types=[
            pltpu.VMEM((B * K,), jnp.int32),
            pltpu.VMEM((E,), jnp.int32),
            pltpu.VMEM((16,), jnp.int32),
        ],
    )(_sc_compact_body)
    ids, nact16 = sc_compact(tki.reshape(B * K))
    nact = nact16[:1]

    shared_out = pl.pallas_call(
        _shared_body,
        out_shape=jax.ShapeDtypeStruct((B, H), jnp.float32),
    )(x, shared_gate_w, shared_up_w, shared_down_w)

    grid_spec = pltpu.PrefetchScalarGridSpec(
        num_scalar_prefetch=2,
        grid=(E,),
        in_specs=[
            pl.BlockSpec((B, H), lambda i, ids, nact: (0, 0)),
            pl.BlockSpec((1, H, I), lambda i, ids, nact: (ids[i], 0, 0)),
            pl.BlockSpec((1, H, I), lambda i, ids, nact: (ids[i], 0, 0)),
            pl.BlockSpec((1, I, H), lambda i, ids, nact: (ids[i], 0, 0)),
            pl.BlockSpec((B, K), lambda i, ids, nact: (0, 0)),
            pl.BlockSpec((B, K), lambda i, ids, nact: (0, 0)),
            pl.BlockSpec((B, H), lambda i, ids, nact: (0, 0)),
        ],
        out_specs=pl.BlockSpec((B, H), lambda i, ids, nact: (0, 0)),
    )
    out = pl.pallas_call(
        _expert_body,
        grid_spec=grid_spec,
        out_shape=jax.ShapeDtypeStruct((B, H), jnp.float32),
        compiler_params=pltpu.CompilerParams(
            dimension_semantics=("arbitrary",),
        ),
    )(ids, nact, x, gate_w, up_w, down_w, tki, tkw, shared_out)

    return out.reshape(B, 1, H)


# speedup vs baseline: 1.0144x; 1.0008x over previous
import functools

import jax
import jax.numpy as jnp
from jax import lax
from jax.experimental import pallas as pl
from jax.experimental.pallas import tpu as pltpu
from jax.experimental.pallas import tpu_sc as plsc

K = 2
SCALE = 2.5
B, H, E, I, SI = 32, 1024, 64, 512, 2048


def _routing_body(x_ref, rw_ref, sgw_ref, suw_ref, sdw_ref,
                  tki_ref, tkw_ref, shared_ref):
    xb = x_ref[...].astype(jnp.bfloat16)
    sg = jnp.dot(xb, sgw_ref[...].astype(jnp.bfloat16),
                 preferred_element_type=jnp.float32)
    su = jnp.dot(xb, suw_ref[...].astype(jnp.bfloat16),
                 preferred_element_type=jnp.float32)
    sact = (sg * jax.lax.logistic(sg) * su).astype(jnp.bfloat16)
    shared_ref[...] = jnp.dot(sact, sdw_ref[...].astype(jnp.bfloat16),
                              preferred_element_type=jnp.float32)

    x = x_ref[...]
    logits = jnp.dot(x, rw_ref[...], preferred_element_type=jnp.float32)
    m = jnp.max(logits, axis=1, keepdims=True)
    p = jnp.exp(logits - m)
    scores = p / jnp.sum(p, axis=1, keepdims=True)

    e_iota = jax.lax.broadcasted_iota(jnp.int32, (B, E), 1)
    m1 = jnp.max(scores, axis=1, keepdims=True)
    i1 = jnp.min(jnp.where(scores == m1, e_iota, E), axis=1, keepdims=True)
    sc2 = jnp.where(e_iota == i1, -jnp.inf, scores)
    m2 = jnp.max(sc2, axis=1, keepdims=True)
    i2 = jnp.min(jnp.where(sc2 == m2, e_iota, E), axis=1, keepdims=True)

    denom = m1 + m2 + 1e-20
    w1 = m1 / denom * SCALE
    w2 = m2 / denom * SCALE
    tki_ref[...] = jnp.concatenate([i1, i2], axis=1)
    tkw_ref[...] = jnp.concatenate([w1, w2], axis=1)


def _sc_compact_body(tki_hbm, ids_hbm, nact_hbm, tki_v, act_v, rank_v,
                     ids_v, nact_v):
    c = lax.axis_index("c")
    s = lax.axis_index("s")

    @pl.when((c == 0) & (s == 0))
    def _():
        pltpu.sync_copy(tki_hbm, tki_v)
        iota = lax.broadcasted_iota(jnp.int32, (16,), 0)
        zero = jnp.zeros((16,), jnp.int32)
        ones = jnp.ones((16,), jnp.int32)

        for ch in range(E // 16):
            act_v[pl.ds(16 * ch, 16)] = zero
        for cs in range(B * K // 16):
            sel = tki_v[pl.ds(16 * cs, 16)]
            plsc.addupdate_scatter(act_v, [sel], ones)

        carry = jnp.int32(0)
        last = jnp.int32(-1)
        for ch in range(E // 16):
            a = jnp.where(act_v[pl.ds(16 * ch, 16)] > 0, 1, 0)
            cs_ = plsc.cumsum(a)
            rank_v[pl.ds(16 * ch, 16)] = cs_ - a + carry
            e_ids = iota + 16 * ch
            last = jnp.maximum(last, jnp.max(jnp.where(a > 0, e_ids, -1)))
            carry = carry + jnp.sum(a)

        for ch in range(E // 16):
            ids_v[pl.ds(16 * ch, 16)] = zero + last
        for ch in range(E // 16):
            a = act_v[pl.ds(16 * ch, 16)] > 0
            rk = rank_v[pl.ds(16 * ch, 16)]
            e_ids = iota + 16 * ch
            plsc.store_scatter(ids_v, [rk], e_ids, mask=a)
        nact_v[...] = zero + carry
        pltpu.sync_copy(ids_v, ids_hbm)
        pltpu.sync_copy(nact_v, nact_hbm)


def _expert_body(ids_ref, nact_ref, x_ref, gw_ref, uw_ref, dw_ref,
                 tki_ref, tkw_ref, shared_ref, out_ref):
    i = pl.program_id(0)

    @pl.when(i == 0)
    def _init():
        out_ref[...] = shared_ref[...]

    @pl.when(i < nact_ref[0])
    def _expert():
        e = ids_ref[i]
        x = x_ref[...].astype(jnp.bfloat16)
        g = jnp.dot(x, gw_ref[0].astype(jnp.bfloat16),
                    preferred_element_type=jnp.float32)
        u = jnp.dot(x, uw_ref[0].astype(jnp.bfloat16),
                    preferred_element_type=jnp.float32)
        a = (g * jax.lax.logistic(g) * u).astype(jnp.bfloat16)
        y = jnp.dot(a, dw_ref[0].astype(jnp.bfloat16),
                    preferred_element_type=jnp.float32)
        coef = jnp.sum(jnp.where(tki_ref[...] == e, tkw_ref[...], 0.0),
                       axis=1, keepdims=True)
        out_ref[...] += y * coef


@jax.jit
def kernel(hidden_states, router_w, e_score_correction_bias, gate_w, up_w,
           down_w, shared_gate_w, shared_up_w, shared_down_w):
    del e_score_correction_bias
    x = hidden_states.reshape(B, H)

    tki, tkw, shared_out = pl.pallas_call(
        _routing_body,
        out_shape=(
            jax.ShapeDtypeStruct((B, K), jnp.int32),
            jax.ShapeDtypeStruct((B, K), jnp.float32),
            jax.ShapeDtypeStruct((B, H), jnp.float32),
        ),
    )(x, router_w, shared_gate_w, shared_up_w, shared_down_w)

    sc_compact = functools.partial(
        pl.kernel,
        out_type=(
            jax.ShapeDtypeStruct((E,), jnp.int32),
            jax.ShapeDtypeStruct((16,), jnp.int32),
        ),
        mesh=plsc.VectorSubcoreMesh(core_axis_name="c", subcore_axis_name="s"),
        compiler_params=pltpu.CompilerParams(needs_layout_passes=False),
        scratch_types=[
            pltpu.VMEM((B * K,), jnp.int32),
            pltpu.VMEM((E,), jnp.int32),
            pltpu.VMEM((E,), jnp.int32),
            pltpu.VMEM((E,), jnp.int32),
            pltpu.VMEM((16,), jnp.int32),
        ],
    )(_sc_compact_body)
    ids, nact16 = sc_compact(tki.reshape(B * K))
    nact = nact16[:1]

    grid_spec = pltpu.PrefetchScalarGridSpec(
        num_scalar_prefetch=2,
        grid=(E,),
        in_specs=[
            pl.BlockSpec((B, H), lambda i, ids, nact: (0, 0)),
            pl.BlockSpec((1, H, I), lambda i, ids, nact: (ids[i], 0, 0)),
            pl.BlockSpec((1, H, I), lambda i, ids, nact: (ids[i], 0, 0)),
            pl.BlockSpec((1, I, H), lambda i, ids, nact: (ids[i], 0, 0)),
            pl.BlockSpec((B, K), lambda i, ids, nact: (0, 0)),
            pl.BlockSpec((B, K), lambda i, ids, nact: (0, 0)),
            pl.BlockSpec((B, H), lambda i, ids, nact: (0, 0)),
        ],
        out_specs=pl.BlockSpec((B, H), lambda i, ids, nact: (0, 0)),
    )
    out = pl.pallas_call(
        _expert_body,
        grid_spec=grid_spec,
        out_shape=jax.ShapeDtypeStruct((B, H), jnp.float32),
        compiler_params=pltpu.CompilerParams(
            dimension_semantics=("arbitrary",),
        ),
    )(ids, nact, x, gate_w, up_w, down_w, tki, tkw, shared_out)

    return out.reshape(B, 1, H)


# speedup vs baseline: 1.0357x; 1.0210x over previous
import functools

import jax
import jax.numpy as jnp
from jax import lax
from jax.experimental import pallas as pl
from jax.experimental.pallas import tpu as pltpu
from jax.experimental.pallas import tpu_sc as plsc

K = 2
SCALE = 2.5
B, H, E, I, SI = 32, 1024, 64, 512, 2048


def _routing_body(x_ref, rw_ref, tki_ref, tkw_ref):
    x = x_ref[...]
    logits = jnp.dot(x, rw_ref[...], preferred_element_type=jnp.float32)
    m = jnp.max(logits, axis=1, keepdims=True)
    p = jnp.exp(logits - m)
    scores = p / jnp.sum(p, axis=1, keepdims=True)

    e_iota = jax.lax.broadcasted_iota(jnp.int32, (B, E), 1)
    m1 = jnp.max(scores, axis=1, keepdims=True)
    i1 = jnp.min(jnp.where(scores == m1, e_iota, E), axis=1, keepdims=True)
    sc2 = jnp.where(e_iota == i1, -jnp.inf, scores)
    m2 = jnp.max(sc2, axis=1, keepdims=True)
    i2 = jnp.min(jnp.where(sc2 == m2, e_iota, E), axis=1, keepdims=True)

    denom = m1 + m2 + 1e-20
    w1 = m1 / denom * SCALE
    w2 = m2 / denom * SCALE
    tki_ref[...] = jnp.concatenate([i1, i2], axis=1)
    tkw_ref[...] = jnp.concatenate([w1, w2], axis=1)


def _sc_compact_body(tki_hbm, ids_hbm, nact_hbm, tki_v, act_v, rank_v,
                     ids_v, nact_v):
    c = lax.axis_index("c")
    s = lax.axis_index("s")

    @pl.when((c == 0) & (s == 0))
    def _():
        pltpu.sync_copy(tki_hbm, tki_v)
        iota = lax.broadcasted_iota(jnp.int32, (16,), 0)
        zero = jnp.zeros((16,), jnp.int32)
        ones = jnp.ones((16,), jnp.int32)

        for ch in range(E // 16):
            act_v[pl.ds(16 * ch, 16)] = zero
        for cs in range(B * K // 16):
            sel = tki_v[pl.ds(16 * cs, 16)]
            plsc.addupdate_scatter(act_v, [sel], ones)

        carry = jnp.int32(0)
        last = jnp.int32(-1)
        for ch in range(E // 16):
            a = jnp.where(act_v[pl.ds(16 * ch, 16)] > 0, 1, 0)
            cs_ = plsc.cumsum(a)
            rank_v[pl.ds(16 * ch, 16)] = cs_ - a + carry
            e_ids = iota + 16 * ch
            last = jnp.maximum(last, jnp.max(jnp.where(a > 0, e_ids, -1)))
            carry = carry + jnp.sum(a)

        for ch in range(E // 16):
            ids_v[pl.ds(16 * ch, 16)] = zero + last
        for ch in range(E // 16):
            a = act_v[pl.ds(16 * ch, 16)] > 0
            rk = rank_v[pl.ds(16 * ch, 16)]
            e_ids = iota + 16 * ch
            plsc.store_scatter(ids_v, [rk], e_ids, mask=a)
        nact_v[...] = zero + carry
        pltpu.sync_copy(ids_v, ids_hbm)
        pltpu.sync_copy(nact_v, nact_hbm)


def _shared_body(x_ref, sgw_ref, suw_ref, sdw_ref, out_ref):
    x = x_ref[...].astype(jnp.bfloat16)
    sg = jnp.dot(x, sgw_ref[...].astype(jnp.bfloat16),
                 preferred_element_type=jnp.float32)
    su = jnp.dot(x, suw_ref[...].astype(jnp.bfloat16),
                 preferred_element_type=jnp.float32)
    act = (sg * jax.lax.logistic(sg) * su).astype(jnp.bfloat16)
    out_ref[...] = jnp.dot(act, sdw_ref[...].astype(jnp.bfloat16),
                           preferred_element_type=jnp.float32)


def _expert_body(ids_ref, nact_ref, x_ref, gw_ref, uw_ref, dw_ref,
                 tki_ref, tkw_ref, shared_ref, out_ref):
    i = pl.program_id(0)

    @pl.when(i == 0)
    def _init():
        out_ref[...] = shared_ref[...]

    @pl.when(i < nact_ref[0])
    def _expert():
        e = ids_ref[i]
        x = x_ref[...].astype(jnp.bfloat16)
        g = jnp.dot(x, gw_ref[0].astype(jnp.bfloat16),
                    preferred_element_type=jnp.float32)
        u = jnp.dot(x, uw_ref[0].astype(jnp.bfloat16),
                    preferred_element_type=jnp.float32)
        a = (g * jax.lax.logistic(g) * u).astype(jnp.bfloat16)
        y = jnp.dot(a, dw_ref[0].astype(jnp.bfloat16),
                    preferred_element_type=jnp.float32)
        coef = jnp.sum(jnp.where(tki_ref[...] == e, tkw_ref[...], 0.0),
                       axis=1, keepdims=True)
        out_ref[...] += y * coef


@jax.jit
def kernel(hidden_states, router_w, e_score_correction_bias, gate_w, up_w,
           down_w, shared_gate_w, shared_up_w, shared_down_w):
    del e_score_correction_bias
    x = hidden_states.reshape(B, H)

    tki, tkw = pl.pallas_call(
        _routing_body,
        out_shape=(
            jax.ShapeDtypeStruct((B, K), jnp.int32),
            jax.ShapeDtypeStruct((B, K), jnp.float32),
        ),
    )(x, router_w)

    sc_compact = functools.partial(
        pl.kernel,
        out_type=(
            jax.ShapeDtypeStruct((E,), jnp.int32),
            jax.ShapeDtypeStruct((16,), jnp.int32),
        ),
        mesh=plsc.VectorSubcoreMesh(core_axis_name="c", subcore_axis_name="s"),
        compiler_params=pltpu.CompilerParams(needs_layout_passes=False),
        scratch_types=[
            pltpu.VMEM((B * K,), jnp.int32),
            pltpu.VMEM((E,), jnp.int32),
            pltpu.VMEM((E,), jnp.int32),
            pltpu.VMEM((E,), jnp.int32),
            pltpu.VMEM((16,), jnp.int32),
        ],
    )(_sc_compact_body)
    ids, nact16 = sc_compact(tki.reshape(B * K))
    nact = nact16[:1]

    shared_out = pl.pallas_call(
        _shared_body,
        out_shape=jax.ShapeDtypeStruct((B, H), jnp.float32),
    )(x, shared_gate_w, shared_up_w, shared_down_w)

    grid_spec = pltpu.PrefetchScalarGridSpec(
        num_scalar_prefetch=2,
        grid=(E,),
        in_specs=[
            pl.BlockSpec((B, H), lambda i, ids, nact: (0, 0)),
            pl.BlockSpec((1, H, I), lambda i, ids, nact: (ids[i], 0, 0)),
            pl.BlockSpec((1, H, I), lambda i, ids, nact: (ids[i], 0, 0)),
            pl.BlockSpec((1, I, H), lambda i, ids, nact: (ids[i], 0, 0)),
            pl.BlockSpec((B, K), lambda i, ids, nact: (0, 0)),
            pl.BlockSpec((B, K), lambda i, ids, nact: (0, 0)),
            pl.BlockSpec((B, H), lambda i, ids, nact: (0, 0)),
        ],
        out_specs=pl.BlockSpec((B, H), lambda i, ids, nact: (0, 0)),
    )
    out = pl.pallas_call(
        _expert_body,
        grid_spec=grid_spec,
        out_shape=jax.ShapeDtypeStruct((B, H), jnp.float32),
        compiler_params=pltpu.CompilerParams(
            dimension_semantics=("arbitrary",),
        ),
    )(ids, nact, x, gate_w, up_w, down_w, tki, tkw, shared_out)

    return out.reshape(B, 1, H)
